# baseline (speedup 1.0000x reference)
import functools

import jax
import jax.numpy as jnp
from jax import lax
from jax.experimental import pallas as pl
from jax.experimental.pallas import tpu as pltpu
from jax.experimental.pallas import tpu_sc as plsc

NAME_NUM = 1000000
SUBJECT_NUM = 128
MAX_LEN = 200
FACTOR_NUM = 64
BATCH = 16384

_INFO = plsc.get_sparse_core_info()
_NC = _INFO.num_cores
_NS = _INFO.num_subcores
_NW = _NC * _NS
_BPW = BATCH // _NW


def _proj_body(sw_ref, w_ref, b_ref, out_ref):
    out_ref[...] = lax.dot_general(
        sw_ref[...], w_ref[...],
        dimension_numbers=(((1,), (1,)), ((), ())),
        preferred_element_type=jnp.float32,
    ) + b_ref[...]


_proj_call = pl.pallas_call(
    _proj_body,
    out_shape=jax.ShapeDtypeStruct((MAX_LEN, FACTOR_NUM), jnp.float32),
)


_mesh = plsc.VectorSubcoreMesh(core_axis_name="c", subcore_axis_name="s")


@functools.partial(
    pl.kernel,
    mesh=_mesh,
    compiler_params=pltpu.CompilerParams(use_tc_tiling_on_sc=False),
    out_type=[
        jax.ShapeDtypeStruct((BATCH, FACTOR_NUM), jnp.float32),
        jax.ShapeDtypeStruct((BATCH, FACTOR_NUM), jnp.float32),
    ],
    scratch_types=[
        pltpu.VMEM((_BPW,), jnp.int32),
        pltpu.VMEM((_BPW,), jnp.int32),
        pltpu.VMEM((_BPW, FACTOR_NUM), jnp.float32),
        pltpu.VMEM((_BPW, FACTOR_NUM), jnp.float32),
        pltpu.SemaphoreType.DMA,
        pltpu.SemaphoreType.DMA,
    ],
)
def _sc_gather2(name_hbm, subj_hbm, table_hbm, proj_hbm, out_n, out_p,
                nidx_v, sidx_v, nrows_v, prows_v, sem_n, sem_p):
    wid = lax.axis_index("s") * _NC + lax.axis_index("c")
    base = wid * _BPW
    pltpu.sync_copy(name_hbm.at[pl.ds(base, _BPW)], nidx_v)
    pltpu.sync_copy(subj_hbm.at[pl.ds(base, _BPW)], sidx_v)
    cp_n = pltpu.async_copy(table_hbm.at[nidx_v], nrows_v, sem_n)
    cp_p = pltpu.async_copy(proj_hbm.at[sidx_v], prows_v, sem_p)
    cp_n.wait()
    cp_p.wait()
    pltpu.sync_copy(nrows_v, out_n.at[pl.ds(base, _BPW)])
    pltpu.sync_copy(prows_v, out_p.at[pl.ds(base, _BPW)])


def _dot_softmax_body(a_ref, b_ref, out_ref):
    row = jnp.sum(a_ref[...] * b_ref[...], axis=1, keepdims=True)
    m = jnp.max(row)
    e = jnp.exp(row - m)
    out_ref[...] = e / jnp.sum(e)


_dot_softmax_call = pl.pallas_call(
    _dot_softmax_body,
    out_shape=jax.ShapeDtypeStruct((BATCH, 1), jnp.float32),
)


def kernel(subject, name, idx, embed_name_w, embed_subject_w, W, b):
    proj_table = _proj_call(embed_subject_w, W, b.reshape(1, FACTOR_NUM))
    name_rows, proj_rows = _sc_gather2(name, subject, embed_name_w, proj_table)
    out = _dot_softmax_call(name_rows, proj_rows)
    return out.reshape(BATCH)


# speedup vs baseline: 1.0073x; 1.0073x over previous
import functools

import jax
import jax.numpy as jnp
from jax import lax
from jax.experimental import pallas as pl
from jax.experimental.pallas import tpu as pltpu
from jax.experimental.pallas import tpu_sc as plsc

NAME_NUM = 1000000
SUBJECT_NUM = 128
MAX_LEN = 200
FACTOR_NUM = 64
BATCH = 16384
_SQ = 128

_INFO = plsc.get_sparse_core_info()
_NC = _INFO.num_cores
_NS = _INFO.num_subcores
_NW = _NC * _NS
_BPW = BATCH // _NW


def _k1_body(subj_ref, par_ref, sw_ref, w_ref, b_ref, out_ref):
    pt = lax.dot_general(
        sw_ref[...], w_ref[...],
        dimension_numbers=(((1,), (1,)), ((), ())),
        preferred_element_type=jnp.float32,
    ) + b_ref[...]
    iota = lax.broadcasted_iota(jnp.int32, (BATCH, MAX_LEN), 1)
    oh = (subj_ref[...] == iota).astype(jnp.float32)
    sp = lax.dot_general(
        oh, pt,
        dimension_numbers=(((1,), (0,)), ((), ())),
        preferred_element_type=jnp.float32,
    )
    par = par_ref[...]
    left = sp * (par == 0).astype(jnp.float32)
    right = sp * (par == 1).astype(jnp.float32)
    out_ref[...] = jnp.concatenate([left, right], axis=1)


_k1_call = pl.pallas_call(
    _k1_body,
    out_shape=jax.ShapeDtypeStruct((BATCH, 2 * FACTOR_NUM), jnp.float32),
)


_mesh = plsc.VectorSubcoreMesh(core_axis_name="c", subcore_axis_name="s")


@functools.partial(
    pl.kernel,
    mesh=_mesh,
    out_type=jax.ShapeDtypeStruct((BATCH, 2 * FACTOR_NUM), jnp.float32),
    scratch_types=[
        pltpu.VMEM((_BPW,), jnp.int32),
        pltpu.VMEM((_BPW, 2 * FACTOR_NUM), jnp.float32),
        pltpu.SemaphoreType.DMA,
    ],
)
def _sc_gather(gidx_hbm, table_hbm, out_rows, gidx_v, rows_v, sem):
    wid = lax.axis_index("s") * _NC + lax.axis_index("c")
    base = wid * _BPW
    pltpu.sync_copy(gidx_hbm.at[pl.ds(base, _BPW)], gidx_v)
    pltpu.async_copy(table_hbm.at[gidx_v], rows_v, sem).wait()
    pltpu.sync_copy(rows_v, out_rows.at[pl.ds(base, _BPW)])


def _p3_body(a_ref, p_ref, out_ref):
    logits = jnp.sum(a_ref[...] * p_ref[...], axis=2)
    m = jnp.max(logits)
    e = jnp.exp(logits - m)
    out_ref[...] = e / jnp.sum(e)


_p3_call = pl.pallas_call(
    _p3_body,
    out_shape=jax.ShapeDtypeStruct((_SQ, _SQ), jnp.float32),
)


def kernel(subject, name, idx, embed_name_w, embed_subject_w, W, b):
    table2 = embed_name_w.reshape(NAME_NUM // 2, 2 * FACTOR_NUM)
    gidx = name >> 1
    parity = name & 1
    prow = _k1_call(
        subject.reshape(BATCH, 1),
        parity.reshape(BATCH, 1),
        embed_subject_w,
        W,
        b.reshape(1, FACTOR_NUM),
    )
    pair_rows = _sc_gather(gidx, table2)
    out = _p3_call(
        pair_rows.reshape(_SQ, _SQ, _SQ),
        prow.reshape(_SQ, _SQ, _SQ),
    )
    return out.reshape(BATCH)


# speedup vs baseline: 1.0107x; 1.0033x over previous
import functools

import jax
import jax.numpy as jnp
from jax import lax
from jax.experimental import pallas as pl
from jax.experimental.pallas import tpu as pltpu
from jax.experimental.pallas import tpu_sc as plsc

NAME_NUM = 1000000
SUBJECT_NUM = 128
MAX_LEN = 200
FACTOR_NUM = 64
BATCH = 16384
_SQ = 128

_INFO = plsc.get_sparse_core_info()
_NC = _INFO.num_cores
_NS = _INFO.num_subcores
_NW = _NC * _NS
_BPW = BATCH // _NW


def _k0_body(sw_ref, w_ref, b_ref, out_ref):
    out_ref[...] = lax.dot_general(
        sw_ref[...], w_ref[...],
        dimension_numbers=(((1,), (1,)), ((), ())),
        preferred_element_type=jnp.float32,
    ) + b_ref[...]


_k0_call = pl.pallas_call(
    _k0_body,
    out_shape=jax.ShapeDtypeStruct((MAX_LEN, FACTOR_NUM), jnp.float32),
)


_mesh = plsc.VectorSubcoreMesh(core_axis_name="c", subcore_axis_name="s")


@functools.partial(
    pl.kernel,
    mesh=_mesh,
    compiler_params=pltpu.CompilerParams(use_tc_tiling_on_sc=False),
    out_type=[
        jax.ShapeDtypeStruct((BATCH, FACTOR_NUM), jnp.float32),
        jax.ShapeDtypeStruct((BATCH, FACTOR_NUM), jnp.float32),
    ],
    scratch_types=[
        pltpu.VMEM((_BPW,), jnp.int32),
        pltpu.VMEM((_BPW,), jnp.int32),
        pltpu.VMEM((_BPW, FACTOR_NUM), jnp.float32),
        pltpu.VMEM((_BPW, FACTOR_NUM), jnp.float32),
        pltpu.SemaphoreType.DMA,
        pltpu.SemaphoreType.DMA,
    ],
)
def _sc_gather2(name_hbm, subj_hbm, table_hbm, proj_hbm, out_n, out_p,
                nidx_v, sidx_v, nrows_v, prows_v, sem_n, sem_p):
    wid = lax.axis_index("s") * _NC + lax.axis_index("c")
    base = wid * _BPW
    pltpu.sync_copy(name_hbm.at[pl.ds(base, _BPW)], nidx_v)
    pltpu.sync_copy(subj_hbm.at[pl.ds(base, _BPW)], sidx_v)
    cp_n = pltpu.async_copy(table_hbm.at[nidx_v], nrows_v, sem_n)
    cp_p = pltpu.async_copy(proj_hbm.at[sidx_v], prows_v, sem_p)
    cp_n.wait()
    cp_p.wait()
    pltpu.sync_copy(nrows_v, out_n.at[pl.ds(base, _BPW)])
    pltpu.sync_copy(prows_v, out_p.at[pl.ds(base, _BPW)])


def _p3_body(a_ref, p_ref, out_ref):
    logits = jnp.sum(a_ref[...] * p_ref[...], axis=2)
    m = jnp.max(logits)
    e = jnp.exp(logits - m)
    out_ref[...] = e / jnp.sum(e)


_p3_call = pl.pallas_call(
    _p3_body,
    out_shape=jax.ShapeDtypeStruct((_SQ, _SQ), jnp.float32),
)


def kernel(subject, name, idx, embed_name_w, embed_subject_w, W, b):
    proj_table = _k0_call(embed_subject_w, W, b.reshape(1, FACTOR_NUM))
    name_rows, proj_rows = _sc_gather2(name, subject, embed_name_w, proj_table)
    out = _p3_call(
        name_rows.reshape(_SQ, _SQ, FACTOR_NUM),
        proj_rows.reshape(_SQ, _SQ, FACTOR_NUM),
    )
    return out.reshape(BATCH)
